# baseline (speedup 1.0000x reference)
import functools

import jax
import jax.numpy as jnp
from jax import lax
from jax.experimental import pallas as pl
from jax.experimental.pallas import tpu as pltpu
from jax.experimental.pallas import tpu_sc as plsc

N = 10000
E = 320000
D = 128
H = 128

NUM_SUBCORES = 16
CH = 128
N_ACC = 10240
ROWS_PER_TILE = N_ACC // NUM_SUBCORES
CHUNKS = 158
E_PER_TILE = CHUNKS * CH
E_PAD = NUM_SUBCORES * E_PER_TILE
PAIRS = CHUNKS // 2
DUMMY = N

ROW_BLK = 512
GRID = N_ACC // ROW_BLK

_MESH = plsc.VectorSubcoreMesh(core_axis_name="c", subcore_axis_name="s")


def _zero_slab(zeros_hbm, slab, row_base, sem):
    n = ROWS_PER_TILE // CH
    for k in range(n):
        pltpu.async_copy(zeros_hbm, slab.at[pl.ds(row_base + k * CH, CH)], sem)
    for k in range(n):
        pltpu.make_async_copy(zeros_hbm,
                              slab.at[pl.ds(row_base + k * CH, CH)], sem).wait()


def _writeback(slab, out_hbm, row_base, sem):
    n = ROWS_PER_TILE // CH
    for k in range(n):
        sl = pl.ds(row_base + k * CH, CH)
        pltpu.async_copy(slab.at[sl], out_hbm.at[sl], sem)
    for k in range(n):
        sl = pl.ds(row_base + k * CH, CH)
        pltpu.make_async_copy(slab.at[sl], out_hbm.at[sl], sem).wait()


HIST_ROWS = N_ACC // 128


def _hist_body(dst_hbm, dstf_hbm, ident_hbm, zeros_hbm, out_hbm, outf_hbm,
               idx_v, ident_v, local_v, table_sh, sem):
    cid = lax.axis_index("c")
    sid = lax.axis_index("s")

    pltpu.sync_copy(zeros_hbm, local_v)
    pltpu.sync_copy(ident_hbm, ident_v)
    pltpu.sync_copy(zeros_hbm.at[pl.ds(0, 8)], table_sh.at[pl.ds(sid * 8, 8)])
    plsc.subcore_barrier()

    def run(edges_hbm):
        pltpu.sync_copy(edges_hbm.at[sid], idx_v)

        def chunk(g, carry):
            for j in range(8):
                idx = idx_v[g, pl.ds(j * 16, 16)]
                row = lax.shift_right_logical(idx, 7)
                col = lax.bitwise_and(idx, 127)
                plsc.addupdate_scatter(local_v, [row, col],
                                       jnp.ones((16,), jnp.float32))
            return carry
        lax.fori_loop(0, CHUNKS, chunk, 0)

    @pl.when(cid == 0)
    def _():
        run(dst_hbm)

    @pl.when(cid == 1)
    def _():
        run(dstf_hbm)

    plsc.subcore_barrier()
    pltpu.sync_copy(local_v, table_sh.at[ident_v], add=True)
    plsc.subcore_barrier()

    @pl.when(cid == 0)
    def _():
        pltpu.sync_copy(table_sh.at[pl.ds(sid * 8, 8)],
                        out_hbm.at[pl.ds(sid * 8, 8)])

    @pl.when(cid == 1)
    def _():
        pltpu.sync_copy(table_sh.at[pl.ds(sid * 8, 8)],
                        outf_hbm.at[pl.ds(sid * 8, 8)])


def _sc_hist(dst_pad, dstf_pad, ident, zeros128):
    return pl.kernel(
        _hist_body,
        out_type=[
            jax.ShapeDtypeStruct((128, 128), jnp.float32),
            jax.ShapeDtypeStruct((128, 128), jnp.float32),
        ],
        mesh=_MESH,
        compiler_params=pltpu.CompilerParams(needs_layout_passes=False),
        scratch_types=[
            pltpu.VMEM((CHUNKS, CH), jnp.int32),
            pltpu.VMEM((128,), jnp.int32),
            pltpu.VMEM((128, 128), jnp.float32),
            pltpu.VMEM_SHARED((128, 128), jnp.float32),
            pltpu.SemaphoreType.DMA,
        ],
    )(dst_pad, dstf_pad, ident, zeros128)


def _spmv_body(u_hbm, uf_hbm, e_hbm, ef_hbm, zeros_hbm,
               out_hbm, outf_hbm, ia_v, ib_v, rows_a, rows_b,
               acc_sh, sem_g, sem_s):
    cid = lax.axis_index("c")
    sid = lax.axis_index("s")
    row_base = sid * ROWS_PER_TILE

    _zero_slab(zeros_hbm, acc_sh, row_base, sem_g)
    plsc.subcore_barrier()

    def run(table_hbm, edges):
        def wait_scatter():
            pltpu.make_async_copy(zeros_hbm, rows_a, sem_s).wait()

        def pair(j, carry):
            a = 2 * j
            pltpu.sync_copy(edges.at[sid, a], ia_v)
            pltpu.async_copy(table_hbm.at[ia_v.at[0]], rows_a, sem_g).wait()

            @pl.when(j > 0)
            def _():
                wait_scatter()
            pltpu.async_copy(rows_a, acc_sh.at[ia_v.at[1]], sem_s, add=True)

            pltpu.sync_copy(edges.at[sid, a + 1], ib_v)
            pltpu.async_copy(table_hbm.at[ib_v.at[0]], rows_b, sem_g).wait()

            wait_scatter()
            pltpu.async_copy(rows_b, acc_sh.at[ib_v.at[1]], sem_s, add=True)
            return carry
        lax.fori_loop(0, PAIRS, pair, 0)
        wait_scatter()

    @pl.when(cid == 0)
    def _():
        run(u_hbm, e_hbm)

    @pl.when(cid == 1)
    def _():
        run(uf_hbm, ef_hbm)

    plsc.subcore_barrier()

    @pl.when(cid == 0)
    def _():
        _writeback(acc_sh, out_hbm, row_base, sem_g)

    @pl.when(cid == 1)
    def _():
        _writeback(acc_sh, outf_hbm, row_base, sem_g)


def _sc_spmv(u_pad, uf_pad, edges, edgesf, zeros128):
    return pl.kernel(
        _spmv_body,
        out_type=[
            jax.ShapeDtypeStruct((N_ACC, H), jnp.float32),
            jax.ShapeDtypeStruct((N_ACC, H), jnp.float32),
        ],
        mesh=_MESH,
        scratch_types=[
            pltpu.VMEM((2, CH), jnp.int32),
            pltpu.VMEM((2, CH), jnp.int32),
            pltpu.VMEM((CH, H), jnp.float32),
            pltpu.VMEM((CH, H), jnp.float32),
            pltpu.VMEM_SHARED((N_ACC, H), jnp.float32),
            pltpu.SemaphoreType.DMA,
            pltpu.SemaphoreType.DMA,
        ],
    )(u_pad, uf_pad, edges, edgesf, zeros128)


def _stage_a_body(hist_ref, histf_ref, x_ref, fx_ref, w1_ref,
                  u1_ref, u1f_ref, dis_ref, disf_ref):
    dis = lax.rsqrt(hist_ref[...] + 1.0)
    disf = lax.rsqrt(histf_ref[...] + 1.0)
    dis_ref[...] = dis
    disf_ref[...] = disf
    u1_ref[...] = jnp.dot(x_ref[...], w1_ref[...],
                          preferred_element_type=jnp.float32) * dis
    u1f_ref[...] = jnp.dot(fx_ref[...], w1_ref[...],
                           preferred_element_type=jnp.float32) * disf


def _stage_a(hist, histf, x, fx, W1):
    row = pl.BlockSpec((ROW_BLK, 1), lambda i: (i, 0))
    mat = pl.BlockSpec((ROW_BLK, D), lambda i: (i, 0))
    full = pl.BlockSpec((D, H), lambda i: (0, 0))
    return pl.pallas_call(
        _stage_a_body,
        grid=(GRID,),
        in_specs=[row, row, mat, mat, full],
        out_specs=[mat, mat, row, row],
        out_shape=[
            jax.ShapeDtypeStruct((N_ACC, H), jnp.float32),
            jax.ShapeDtypeStruct((N_ACC, H), jnp.float32),
            jax.ShapeDtypeStruct((N_ACC, 1), jnp.float32),
            jax.ShapeDtypeStruct((N_ACC, 1), jnp.float32),
        ],
    )(hist, histf, x, fx, W1)


def _stage_b_body(acc_ref, u_ref, dis_ref, accf_ref, uf_ref, disf_ref,
                  b1_ref, w2_ref, u2_ref, u2f_ref):
    h1 = jax.nn.relu(dis_ref[...] * (acc_ref[...] + u_ref[...]) + b1_ref[...])
    u2_ref[...] = jnp.dot(h1, w2_ref[...],
                          preferred_element_type=jnp.float32) * dis_ref[...]
    h1f = jax.nn.relu(disf_ref[...] * (accf_ref[...] + uf_ref[...]) + b1_ref[...])
    u2f_ref[...] = jnp.dot(h1f, w2_ref[...],
                           preferred_element_type=jnp.float32) * disf_ref[...]


def _stage_b(acc, u, dis, accf, uf, disf, b1, W2):
    row = pl.BlockSpec((ROW_BLK, 1), lambda i: (i, 0))
    mat = pl.BlockSpec((ROW_BLK, H), lambda i: (i, 0))
    vec = pl.BlockSpec((1, H), lambda i: (0, 0))
    full = pl.BlockSpec((H, H), lambda i: (0, 0))
    return pl.pallas_call(
        _stage_b_body,
        grid=(GRID,),
        in_specs=[mat, mat, row, mat, mat, row, vec, full],
        out_specs=[mat, mat],
        out_shape=[
            jax.ShapeDtypeStruct((N_ACC, H), jnp.float32),
            jax.ShapeDtypeStruct((N_ACC, H), jnp.float32),
        ],
    )(acc, u, dis, accf, uf, disf, b1.reshape(1, H), W2)


def _stage_c_body(acc_ref, u_ref, dis_ref, accf_ref, uf_ref, disf_ref,
                  b2_ref, wh_ref, bh_ref, heads_ref, headsf_ref):
    z2 = dis_ref[...] * (acc_ref[...] + u_ref[...]) + b2_ref[...]
    heads_ref[...] = jax.nn.relu(
        jnp.dot(z2, wh_ref[...], preferred_element_type=jnp.float32)
        + bh_ref[...])
    z2f = disf_ref[...] * (accf_ref[...] + uf_ref[...]) + b2_ref[...]
    headsf_ref[...] = jax.nn.relu(
        jnp.dot(z2f, wh_ref[...], preferred_element_type=jnp.float32)
        + bh_ref[...])


def _stage_c(acc, u, dis, accf, uf, disf, b2, Wh, bh):
    row = pl.BlockSpec((ROW_BLK, 1), lambda i: (i, 0))
    mat = pl.BlockSpec((ROW_BLK, H), lambda i: (i, 0))
    vec = pl.BlockSpec((1, H), lambda i: (0, 0))
    wh_spec = pl.BlockSpec((H, 3), lambda i: (0, 0))
    bh_spec = pl.BlockSpec((1, 3), lambda i: (0, 0))
    out3 = pl.BlockSpec((ROW_BLK, 3), lambda i: (i, 0))
    return pl.pallas_call(
        _stage_c_body,
        grid=(GRID,),
        in_specs=[mat, mat, row, mat, mat, row, vec, wh_spec, bh_spec],
        out_specs=[out3, out3],
        out_shape=[
            jax.ShapeDtypeStruct((N_ACC, 3), jnp.float32),
            jax.ShapeDtypeStruct((N_ACC, 3), jnp.float32),
        ],
    )(acc, u, dis, accf, uf, disf, b2.reshape(1, H), Wh, bh.reshape(1, 3))


def _pad_edges(e):
    pad = jnp.full((E_PAD - E,), DUMMY, dtype=jnp.int32)
    return jnp.concatenate([e, pad]).reshape(NUM_SUBCORES, CHUNKS, CH)


def kernel(x, edge_index, fake_x, fake_edge_index, W1, b1, W2, b2,
           Wy, by, Wp, bp, Wb, bb):
    src = _pad_edges(edge_index[0])
    dst = _pad_edges(edge_index[1])
    srcf = _pad_edges(fake_edge_index[0])
    dstf = _pad_edges(fake_edge_index[1])
    edges = jnp.stack([src, dst], axis=2)
    edgesf = jnp.stack([srcf, dstf], axis=2)

    zeros128 = jnp.zeros((CH, H), jnp.float32)
    ident = jnp.concatenate([
        jnp.arange(HIST_ROWS, dtype=jnp.int32),
        jnp.full((128 - HIST_ROWS,), HIST_ROWS, jnp.int32),
    ])

    hist, histf = _sc_hist(dst, dstf, ident, zeros128)
    hist = hist.reshape(N_ACC + 6144, 1)[:N_ACC]
    histf = histf.reshape(N_ACC + 6144, 1)[:N_ACC]

    u1, u1f, dis, disf = _stage_a(hist, histf, x, fake_x, W1)

    acc1, acc1f = _sc_spmv(u1, u1f, edges, edgesf, zeros128)

    u2, u2f = _stage_b(acc1, u1, dis, acc1f, u1f, disf, b1, W2)

    acc2, acc2f = _sc_spmv(u2, u2f, edges, edgesf, zeros128)

    Wh = jnp.concatenate([Wy, Wp, Wb], axis=1)
    bh = jnp.stack([by[0], bp[0], bb[0]])
    heads, headsf = _stage_c(acc2, u2, dis, acc2f, u2f, disf, b2, Wh, bh)

    yi = heads[:N, 0:1]
    fact_prob = heads[:N, 1:2]
    treat_prob = heads[:N, 2:3]
    fact_prob_f = headsf[:N, 1:2]
    return (yi, fact_prob, fact_prob_f, treat_prob)


# speedup vs baseline: 1.0250x; 1.0250x over previous
import functools

import jax
import jax.numpy as jnp
from jax import lax
from jax.experimental import pallas as pl
from jax.experimental.pallas import tpu as pltpu
from jax.experimental.pallas import tpu_sc as plsc

N = 10000
E = 320000
D = 128
H = 128

NUM_SUBCORES = 16
CH = 128
N_ACC = 10240
ROWS_PER_TILE = N_ACC // NUM_SUBCORES
CHUNKS = 157
E_PER_TILE = CHUNKS * CH
E_PAD = NUM_SUBCORES * E_PER_TILE
DUMMY = N

ROW_BLK = 512
GRID = N_ACC // ROW_BLK

_MESH = plsc.VectorSubcoreMesh(core_axis_name="c", subcore_axis_name="s")


def _zero_slab(zeros_hbm, slab, row_base, sem):
    n = ROWS_PER_TILE // CH
    for k in range(n):
        pltpu.async_copy(zeros_hbm, slab.at[pl.ds(row_base + k * CH, CH)], sem)
    for k in range(n):
        pltpu.make_async_copy(zeros_hbm,
                              slab.at[pl.ds(row_base + k * CH, CH)], sem).wait()


def _writeback(slab, out_hbm, row_base, sem):
    n = ROWS_PER_TILE // CH
    for k in range(n):
        sl = pl.ds(row_base + k * CH, CH)
        pltpu.async_copy(slab.at[sl], out_hbm.at[sl], sem)
    for k in range(n):
        sl = pl.ds(row_base + k * CH, CH)
        pltpu.make_async_copy(slab.at[sl], out_hbm.at[sl], sem).wait()


HIST_ROWS = N_ACC // 128


def _hist_body(dst_hbm, dstf_hbm, ident_hbm, zeros_hbm, out_hbm, outf_hbm,
               idx_v, ident_v, local_v, table_sh, sem):
    cid = lax.axis_index("c")
    sid = lax.axis_index("s")

    pltpu.sync_copy(zeros_hbm, local_v)
    pltpu.sync_copy(ident_hbm, ident_v)
    pltpu.sync_copy(zeros_hbm.at[pl.ds(0, 8)], table_sh.at[pl.ds(sid * 8, 8)])
    plsc.subcore_barrier()

    def run(edges_hbm):
        pltpu.sync_copy(edges_hbm.at[sid], idx_v)

        def chunk(g, carry):
            for j in range(8):
                idx = idx_v[g, pl.ds(j * 16, 16)]
                row = lax.shift_right_logical(idx, 7)
                col = lax.bitwise_and(idx, 127)
                plsc.addupdate_scatter(local_v, [row, col],
                                       jnp.ones((16,), jnp.float32))
            return carry
        lax.fori_loop(0, CHUNKS, chunk, 0)

    @pl.when(cid == 0)
    def _():
        run(dst_hbm)

    @pl.when(cid == 1)
    def _():
        run(dstf_hbm)

    plsc.subcore_barrier()
    pltpu.sync_copy(local_v, table_sh.at[ident_v], add=True)
    plsc.subcore_barrier()

    @pl.when(cid == 0)
    def _():
        pltpu.sync_copy(table_sh.at[pl.ds(sid * 8, 8)],
                        out_hbm.at[pl.ds(sid * 8, 8)])

    @pl.when(cid == 1)
    def _():
        pltpu.sync_copy(table_sh.at[pl.ds(sid * 8, 8)],
                        outf_hbm.at[pl.ds(sid * 8, 8)])


def _sc_hist(dst_pad, dstf_pad, ident, zeros128):
    return pl.kernel(
        _hist_body,
        out_type=[
            jax.ShapeDtypeStruct((128, 128), jnp.float32),
            jax.ShapeDtypeStruct((128, 128), jnp.float32),
        ],
        mesh=_MESH,
        compiler_params=pltpu.CompilerParams(needs_layout_passes=False),
        scratch_types=[
            pltpu.VMEM((CHUNKS, CH), jnp.int32),
            pltpu.VMEM((128,), jnp.int32),
            pltpu.VMEM((128, 128), jnp.float32),
            pltpu.VMEM_SHARED((128, 128), jnp.float32),
            pltpu.SemaphoreType.DMA,
        ],
    )(dst_pad, dstf_pad, ident, zeros128)


def _spmv_body(u_hbm, uf_hbm, e_hbm, ef_hbm, zeros_hbm,
               out_hbm, outf_hbm, idx2_v, rows_a, acc_sh, sem_a):
    cid = lax.axis_index("c")
    sid = lax.axis_index("s")
    row_base = sid * ROWS_PER_TILE

    _zero_slab(zeros_hbm, acc_sh, row_base, sem_a)
    plsc.subcore_barrier()

    def run(table_hbm, edges):
        def chunk(g, carry):
            pltpu.sync_copy(edges.at[sid, g], idx2_v)
            pltpu.async_copy(table_hbm.at[idx2_v.at[0]], rows_a, sem_a).wait()
            pltpu.sync_copy(rows_a, acc_sh.at[idx2_v.at[1]], add=True)
            return carry
        lax.fori_loop(0, CHUNKS, chunk, 0)

    @pl.when(cid == 0)
    def _():
        run(u_hbm, e_hbm)

    @pl.when(cid == 1)
    def _():
        run(uf_hbm, ef_hbm)

    plsc.subcore_barrier()

    @pl.when(cid == 0)
    def _():
        _writeback(acc_sh, out_hbm, row_base, sem_a)

    @pl.when(cid == 1)
    def _():
        _writeback(acc_sh, outf_hbm, row_base, sem_a)


def _sc_spmv(u_pad, uf_pad, edges, edgesf, zeros128):
    return pl.kernel(
        _spmv_body,
        out_type=[
            jax.ShapeDtypeStruct((N_ACC, H), jnp.float32),
            jax.ShapeDtypeStruct((N_ACC, H), jnp.float32),
        ],
        mesh=_MESH,
        scratch_types=[
            pltpu.VMEM((2, CH), jnp.int32),
            pltpu.VMEM((CH, H), jnp.float32),
            pltpu.VMEM_SHARED((N_ACC, H), jnp.float32),
            pltpu.SemaphoreType.DMA,
        ],
    )(u_pad, uf_pad, edges, edgesf, zeros128)


def _stage_a_body(hist_ref, histf_ref, x_ref, fx_ref, w1_ref,
                  u1_ref, u1f_ref, dis_ref, disf_ref):
    dis = lax.rsqrt(hist_ref[...] + 1.0)
    disf = lax.rsqrt(histf_ref[...] + 1.0)
    dis_ref[...] = dis
    disf_ref[...] = disf
    u1_ref[...] = jnp.dot(x_ref[...], w1_ref[...],
                          preferred_element_type=jnp.float32) * dis
    u1f_ref[...] = jnp.dot(fx_ref[...], w1_ref[...],
                           preferred_element_type=jnp.float32) * disf


def _stage_a(hist, histf, x, fx, W1):
    row = pl.BlockSpec((ROW_BLK, 1), lambda i: (i, 0))
    mat = pl.BlockSpec((ROW_BLK, D), lambda i: (i, 0))
    full = pl.BlockSpec((D, H), lambda i: (0, 0))
    return pl.pallas_call(
        _stage_a_body,
        grid=(GRID,),
        in_specs=[row, row, mat, mat, full],
        out_specs=[mat, mat, row, row],
        out_shape=[
            jax.ShapeDtypeStruct((N_ACC, H), jnp.float32),
            jax.ShapeDtypeStruct((N_ACC, H), jnp.float32),
            jax.ShapeDtypeStruct((N_ACC, 1), jnp.float32),
            jax.ShapeDtypeStruct((N_ACC, 1), jnp.float32),
        ],
    )(hist, histf, x, fx, W1)


def _stage_b_body(acc_ref, u_ref, dis_ref, accf_ref, uf_ref, disf_ref,
                  b1_ref, w2_ref, u2_ref, u2f_ref):
    h1 = jax.nn.relu(dis_ref[...] * (acc_ref[...] + u_ref[...]) + b1_ref[...])
    u2_ref[...] = jnp.dot(h1, w2_ref[...],
                          preferred_element_type=jnp.float32) * dis_ref[...]
    h1f = jax.nn.relu(disf_ref[...] * (accf_ref[...] + uf_ref[...]) + b1_ref[...])
    u2f_ref[...] = jnp.dot(h1f, w2_ref[...],
                           preferred_element_type=jnp.float32) * disf_ref[...]


def _stage_b(acc, u, dis, accf, uf, disf, b1, W2):
    row = pl.BlockSpec((ROW_BLK, 1), lambda i: (i, 0))
    mat = pl.BlockSpec((ROW_BLK, H), lambda i: (i, 0))
    vec = pl.BlockSpec((1, H), lambda i: (0, 0))
    full = pl.BlockSpec((H, H), lambda i: (0, 0))
    return pl.pallas_call(
        _stage_b_body,
        grid=(GRID,),
        in_specs=[mat, mat, row, mat, mat, row, vec, full],
        out_specs=[mat, mat],
        out_shape=[
            jax.ShapeDtypeStruct((N_ACC, H), jnp.float32),
            jax.ShapeDtypeStruct((N_ACC, H), jnp.float32),
        ],
    )(acc, u, dis, accf, uf, disf, b1.reshape(1, H), W2)


def _stage_c_body(acc_ref, u_ref, dis_ref, accf_ref, uf_ref, disf_ref,
                  b2_ref, wh_ref, bh_ref, heads_ref, headsf_ref):
    z2 = dis_ref[...] * (acc_ref[...] + u_ref[...]) + b2_ref[...]
    heads_ref[...] = jax.nn.relu(
        jnp.dot(z2, wh_ref[...], preferred_element_type=jnp.float32)
        + bh_ref[...])
    z2f = disf_ref[...] * (accf_ref[...] + uf_ref[...]) + b2_ref[...]
    headsf_ref[...] = jax.nn.relu(
        jnp.dot(z2f, wh_ref[...], preferred_element_type=jnp.float32)
        + bh_ref[...])


def _stage_c(acc, u, dis, accf, uf, disf, b2, Wh, bh):
    row = pl.BlockSpec((ROW_BLK, 1), lambda i: (i, 0))
    mat = pl.BlockSpec((ROW_BLK, H), lambda i: (i, 0))
    vec = pl.BlockSpec((1, H), lambda i: (0, 0))
    wh_spec = pl.BlockSpec((H, 3), lambda i: (0, 0))
    bh_spec = pl.BlockSpec((1, 3), lambda i: (0, 0))
    out3 = pl.BlockSpec((ROW_BLK, 3), lambda i: (i, 0))
    return pl.pallas_call(
        _stage_c_body,
        grid=(GRID,),
        in_specs=[mat, mat, row, mat, mat, row, vec, wh_spec, bh_spec],
        out_specs=[out3, out3],
        out_shape=[
            jax.ShapeDtypeStruct((N_ACC, 3), jnp.float32),
            jax.ShapeDtypeStruct((N_ACC, 3), jnp.float32),
        ],
    )(acc, u, dis, accf, uf, disf, b2.reshape(1, H), Wh, bh.reshape(1, 3))


def _pad_edges(e):
    pad = jnp.full((E_PAD - E,), DUMMY, dtype=jnp.int32)
    return jnp.concatenate([e, pad]).reshape(NUM_SUBCORES, CHUNKS, CH)


def kernel(x, edge_index, fake_x, fake_edge_index, W1, b1, W2, b2,
           Wy, by, Wp, bp, Wb, bb):
    src = _pad_edges(edge_index[0])
    dst = _pad_edges(edge_index[1])
    srcf = _pad_edges(fake_edge_index[0])
    dstf = _pad_edges(fake_edge_index[1])
    edges = jnp.stack([src, dst], axis=2)
    edgesf = jnp.stack([srcf, dstf], axis=2)

    zeros128 = jnp.zeros((CH, H), jnp.float32)
    ident = jnp.concatenate([
        jnp.arange(HIST_ROWS, dtype=jnp.int32),
        jnp.full((128 - HIST_ROWS,), HIST_ROWS, jnp.int32),
    ])

    hist, histf = _sc_hist(dst, dstf, ident, zeros128)
    hist = hist.reshape(N_ACC + 6144, 1)[:N_ACC]
    histf = histf.reshape(N_ACC + 6144, 1)[:N_ACC]

    u1, u1f, dis, disf = _stage_a(hist, histf, x, fake_x, W1)

    acc1, acc1f = _sc_spmv(u1, u1f, edges, edgesf, zeros128)

    u2, u2f = _stage_b(acc1, u1, dis, acc1f, u1f, disf, b1, W2)

    acc2, acc2f = _sc_spmv(u2, u2f, edges, edgesf, zeros128)

    Wh = jnp.concatenate([Wy, Wp, Wb], axis=1)
    bh = jnp.stack([by[0], bp[0], bb[0]])
    heads, headsf = _stage_c(acc2, u2, dis, acc2f, u2f, disf, b2, Wh, bh)

    yi = heads[:N, 0:1]
    fact_prob = heads[:N, 1:2]
    treat_prob = heads[:N, 2:3]
    fact_prob_f = headsf[:N, 1:2]
    return (yi, fact_prob, fact_prob_f, treat_prob)


# speedup vs baseline: 1.0657x; 1.0397x over previous
import functools

import jax
import jax.numpy as jnp
from jax import lax
from jax.experimental import pallas as pl
from jax.experimental.pallas import tpu as pltpu
from jax.experimental.pallas import tpu_sc as plsc

N = 10000
E = 320000
D = 128
H = 128

NUM_SUBCORES = 16
CH = 128
N_ACC = 10240
ROWS_PER_TILE = N_ACC // NUM_SUBCORES
CHUNKS = 157
E_PER_TILE = CHUNKS * CH
E_PAD = NUM_SUBCORES * E_PER_TILE
DUMMY = N

ROW_BLK = 512
GRID = N_ACC // ROW_BLK

_MESH = plsc.VectorSubcoreMesh(core_axis_name="c", subcore_axis_name="s")


def _zero_slab(zeros_hbm, slab, row_base, sem):
    n = ROWS_PER_TILE // CH
    for k in range(n):
        pltpu.async_copy(zeros_hbm, slab.at[pl.ds(row_base + k * CH, CH)], sem)
    for k in range(n):
        pltpu.make_async_copy(zeros_hbm,
                              slab.at[pl.ds(row_base + k * CH, CH)], sem).wait()


def _writeback(slab, out_hbm, row_base, sem):
    n = ROWS_PER_TILE // CH
    for k in range(n):
        sl = pl.ds(row_base + k * CH, CH)
        pltpu.async_copy(slab.at[sl], out_hbm.at[sl], sem)
    for k in range(n):
        sl = pl.ds(row_base + k * CH, CH)
        pltpu.make_async_copy(slab.at[sl], out_hbm.at[sl], sem).wait()


HIST_ROWS = N_ACC // 128


def _hist_body(dst_hbm, dstf_hbm, ident_hbm, zeros_hbm, out_hbm, outf_hbm,
               idx_v, ident_v, local_v, table_sh, sem):
    cid = lax.axis_index("c")
    sid = lax.axis_index("s")

    pltpu.sync_copy(zeros_hbm, local_v)
    pltpu.sync_copy(ident_hbm, ident_v)
    pltpu.sync_copy(zeros_hbm.at[pl.ds(0, 8)], table_sh.at[pl.ds(sid * 8, 8)])
    plsc.subcore_barrier()

    def run(edges_hbm):
        pltpu.sync_copy(edges_hbm.at[sid], idx_v)

        def chunk(g, carry):
            for j in range(8):
                idx = idx_v[g, pl.ds(j * 16, 16)]
                row = lax.shift_right_logical(idx, 7)
                col = lax.bitwise_and(idx, 127)
                plsc.addupdate_scatter(local_v, [row, col],
                                       jnp.ones((16,), jnp.float32))
            return carry
        lax.fori_loop(0, CHUNKS, chunk, 0)

    @pl.when(cid == 0)
    def _():
        run(dst_hbm)

    @pl.when(cid == 1)
    def _():
        run(dstf_hbm)

    plsc.subcore_barrier()
    pltpu.sync_copy(local_v, table_sh.at[ident_v], add=True)
    plsc.subcore_barrier()

    @pl.when(cid == 0)
    def _():
        pltpu.sync_copy(table_sh.at[pl.ds(sid * 8, 8)],
                        out_hbm.at[pl.ds(sid * 8, 8)])

    @pl.when(cid == 1)
    def _():
        pltpu.sync_copy(table_sh.at[pl.ds(sid * 8, 8)],
                        outf_hbm.at[pl.ds(sid * 8, 8)])


def _sc_hist(dst_pad, dstf_pad, ident, zeros128):
    return pl.kernel(
        _hist_body,
        out_type=[
            jax.ShapeDtypeStruct((128, 128), jnp.float32),
            jax.ShapeDtypeStruct((128, 128), jnp.float32),
        ],
        mesh=_MESH,
        compiler_params=pltpu.CompilerParams(needs_layout_passes=False),
        scratch_types=[
            pltpu.VMEM((CHUNKS, CH), jnp.int32),
            pltpu.VMEM((128,), jnp.int32),
            pltpu.VMEM((128, 128), jnp.float32),
            pltpu.VMEM_SHARED((128, 128), jnp.float32),
            pltpu.SemaphoreType.DMA,
        ],
    )(dst_pad, dstf_pad, ident, zeros128)


def _spmv_body(u_hbm, uf_hbm, e_hbm, ef_hbm, zeros_hbm,
               out_hbm, outf_hbm, idx2_v, rows_a, acc_sh, sem_a):
    cid = lax.axis_index("c")
    sid = lax.axis_index("s")
    row_base = sid * ROWS_PER_TILE

    _zero_slab(zeros_hbm, acc_sh, row_base, sem_a)
    plsc.subcore_barrier()

    def run(table_hbm, edges):
        def chunk(g, carry):
            pltpu.sync_copy(edges.at[sid, g], idx2_v)
            pltpu.async_copy(table_hbm.at[idx2_v.at[0]], rows_a, sem_a).wait()
            pltpu.sync_copy(rows_a, acc_sh.at[idx2_v.at[1]], add=True)
            return carry
        lax.fori_loop(0, CHUNKS, chunk, 0)

    @pl.when(cid == 0)
    def _():
        run(u_hbm, e_hbm)

    @pl.when(cid == 1)
    def _():
        run(uf_hbm, ef_hbm)

    plsc.subcore_barrier()

    @pl.when(cid == 0)
    def _():
        _writeback(acc_sh, out_hbm, row_base, sem_a)

    @pl.when(cid == 1)
    def _():
        _writeback(acc_sh, outf_hbm, row_base, sem_a)


def _sc_spmv(u_pad, uf_pad, edges, edgesf, zeros128):
    return pl.kernel(
        _spmv_body,
        out_type=[
            jax.ShapeDtypeStruct((N_ACC, H), jnp.float32),
            jax.ShapeDtypeStruct((N_ACC, H), jnp.float32),
        ],
        mesh=_MESH,
        scratch_types=[
            pltpu.VMEM((2, CH), jnp.int32),
            pltpu.VMEM((CH, H), jnp.float32),
            pltpu.VMEM_SHARED((N_ACC, H), jnp.float32),
            pltpu.SemaphoreType.DMA,
        ],
    )(u_pad, uf_pad, edges, edgesf, zeros128)


def _stage_a_body(hist_ref, histf_ref, x_ref, fx_ref, w1_ref,
                  u1_ref, u1f_ref, dis_ref, disf_ref):
    dis = lax.rsqrt(hist_ref[...] + 1.0)
    disf = lax.rsqrt(histf_ref[...] + 1.0)
    dis_ref[...] = dis
    disf_ref[...] = disf
    u1_ref[...] = jnp.dot(x_ref[...], w1_ref[...],
                          preferred_element_type=jnp.float32) * dis
    u1f_ref[...] = jnp.dot(fx_ref[...], w1_ref[...],
                           preferred_element_type=jnp.float32) * disf


def _stage_a(hist, histf, x, fx, W1):
    row = pl.BlockSpec((ROW_BLK, 1), lambda i: (i, 0))
    mat = pl.BlockSpec((ROW_BLK, D), lambda i: (i, 0))
    full = pl.BlockSpec((D, H), lambda i: (0, 0))
    return pl.pallas_call(
        _stage_a_body,
        grid=(GRID,),
        in_specs=[row, row, mat, mat, full],
        out_specs=[mat, mat, row, row],
        out_shape=[
            jax.ShapeDtypeStruct((N_ACC, H), jnp.float32),
            jax.ShapeDtypeStruct((N_ACC, H), jnp.float32),
            jax.ShapeDtypeStruct((N_ACC, 1), jnp.float32),
            jax.ShapeDtypeStruct((N_ACC, 1), jnp.float32),
        ],
    )(hist, histf, x, fx, W1)


def _stage_b_body(acc_ref, u_ref, dis_ref, accf_ref, uf_ref, disf_ref,
                  b1_ref, w2_ref, u2_ref, u2f_ref):
    h1 = jax.nn.relu(dis_ref[...] * (acc_ref[...] + u_ref[...]) + b1_ref[...])
    u2_ref[...] = jnp.dot(h1, w2_ref[...],
                          preferred_element_type=jnp.float32) * dis_ref[...]
    h1f = jax.nn.relu(disf_ref[...] * (accf_ref[...] + uf_ref[...]) + b1_ref[...])
    u2f_ref[...] = jnp.dot(h1f, w2_ref[...],
                           preferred_element_type=jnp.float32) * disf_ref[...]


def _stage_b(acc, u, dis, accf, uf, disf, b1, W2):
    row = pl.BlockSpec((ROW_BLK, 1), lambda i: (i, 0))
    mat = pl.BlockSpec((ROW_BLK, H), lambda i: (i, 0))
    vec = pl.BlockSpec((1, H), lambda i: (0, 0))
    full = pl.BlockSpec((H, H), lambda i: (0, 0))
    return pl.pallas_call(
        _stage_b_body,
        grid=(GRID,),
        in_specs=[mat, mat, row, mat, mat, row, vec, full],
        out_specs=[mat, mat],
        out_shape=[
            jax.ShapeDtypeStruct((N_ACC, H), jnp.float32),
            jax.ShapeDtypeStruct((N_ACC, H), jnp.float32),
        ],
    )(acc, u, dis, accf, uf, disf, b1.reshape(1, H), W2)


def _stage_c_body(acc_ref, u_ref, dis_ref, accf_ref, uf_ref, disf_ref,
                  b2_ref, wh_ref, bh_ref, heads_ref, headsf_ref):
    z2 = dis_ref[...] * (acc_ref[...] + u_ref[...]) + b2_ref[...]
    heads_ref[...] = jax.nn.relu(
        jnp.dot(z2, wh_ref[...], preferred_element_type=jnp.float32)
        + bh_ref[...])
    z2f = disf_ref[...] * (accf_ref[...] + uf_ref[...]) + b2_ref[...]
    headsf_ref[...] = jax.nn.relu(
        jnp.dot(z2f, wh_ref[...], preferred_element_type=jnp.float32)
        + bh_ref[...])


def _stage_c(acc, u, dis, accf, uf, disf, b2, Wh, bh):
    row = pl.BlockSpec((ROW_BLK, 1), lambda i: (i, 0))
    mat = pl.BlockSpec((ROW_BLK, H), lambda i: (i, 0))
    vec = pl.BlockSpec((1, H), lambda i: (0, 0))
    wh_spec = pl.BlockSpec((H, 3), lambda i: (0, 0))
    bh_spec = pl.BlockSpec((1, 3), lambda i: (0, 0))
    out3 = pl.BlockSpec((ROW_BLK, 3), lambda i: (i, 0))
    return pl.pallas_call(
        _stage_c_body,
        grid=(GRID,),
        in_specs=[mat, mat, row, mat, mat, row, vec, wh_spec, bh_spec],
        out_specs=[out3, out3],
        out_shape=[
            jax.ShapeDtypeStruct((N_ACC, 3), jnp.float32),
            jax.ShapeDtypeStruct((N_ACC, 3), jnp.float32),
        ],
    )(acc, u, dis, accf, uf, disf, b2.reshape(1, H), Wh, bh.reshape(1, 3))


def _pad_edges(e):
    per_tile = E // NUM_SUBCORES
    e2 = e.reshape(NUM_SUBCORES, per_tile)
    e2 = jnp.pad(e2, ((0, 0), (0, E_PER_TILE - per_tile)),
                 constant_values=DUMMY)
    return e2.reshape(NUM_SUBCORES, CHUNKS, CH)


def kernel(x, edge_index, fake_x, fake_edge_index, W1, b1, W2, b2,
           Wy, by, Wp, bp, Wb, bb):
    src = _pad_edges(edge_index[0])
    dst = _pad_edges(edge_index[1])
    srcf = _pad_edges(fake_edge_index[0])
    dstf = _pad_edges(fake_edge_index[1])
    edges = jnp.stack([src, dst], axis=2)
    edgesf = jnp.stack([srcf, dstf], axis=2)

    zeros128 = jnp.zeros((CH, H), jnp.float32)
    ident = jnp.concatenate([
        jnp.arange(HIST_ROWS, dtype=jnp.int32),
        jnp.full((128 - HIST_ROWS,), HIST_ROWS, jnp.int32),
    ])

    hist, histf = _sc_hist(dst, dstf, ident, zeros128)
    hist = hist.reshape(N_ACC + 6144, 1)[:N_ACC]
    histf = histf.reshape(N_ACC + 6144, 1)[:N_ACC]

    u1, u1f, dis, disf = _stage_a(hist, histf, x, fake_x, W1)

    acc1, acc1f = _sc_spmv(u1, u1f, edges, edgesf, zeros128)

    u2, u2f = _stage_b(acc1, u1, dis, acc1f, u1f, disf, b1, W2)

    acc2, acc2f = _sc_spmv(u2, u2f, edges, edgesf, zeros128)

    Wh = jnp.concatenate([Wy, Wp, Wb], axis=1)
    bh = jnp.stack([by[0], bp[0], bb[0]])
    heads, headsf = _stage_c(acc2, u2, dis, acc2f, u2f, disf, b2, Wh, bh)

    yi = heads[:N, 0:1]
    fact_prob = heads[:N, 1:2]
    treat_prob = heads[:N, 2:3]
    fact_prob_f = headsf[:N, 1:2]
    return (yi, fact_prob, fact_prob_f, treat_prob)
